# baseline (speedup 1.0000x reference)
import functools

import jax
import jax.numpy as jnp
from jax import lax
from jax.experimental import pallas as pl
from jax.experimental.pallas import tpu as pltpu
from jax.experimental.pallas import tpu_sc as plsc

N_NODES = 10000
HIDDEN = 64
N_LABELS = 64
ALPHA = 0.1
ITERATIONS_ = 10

NC = 2
NS = 16
NW = NC * NS
BLK = 128
NPAD = 10240
ROWS_PER_TILE = NPAD // NS
LANES = 16
CGRP = HIDDEN // LANES


ROWB = 4
IDXB = 8
LOOK_G = 1
LOOK_I = 4


def _make_seg_kernel(e_pad, trows, dual, teleport):
    ew = e_pad // NW
    nblk = ew // BLK
    assert nblk % IDXB == 0
    mesh = plsc.VectorSubcoreMesh(core_axis_name="c", subcore_axis_name="s")

    @functools.partial(
        pl.kernel,
        mesh=mesh,
        compiler_params=pltpu.CompilerParams(
            needs_layout_passes=False, use_tc_tiling_on_sc=False),
        out_type=jax.ShapeDtypeStruct((NC, NPAD, HIDDEN), jnp.float32),
        scratch_types=[
            pltpu.VMEM_SHARED((NPAD, HIDDEN), jnp.float32),
            pltpu.VMEM_SHARED((trows, HIDDEN), jnp.float32),
            pltpu.VMEM((IDXB, BLK), jnp.int32),
            pltpu.VMEM((IDXB, BLK), jnp.int32),
            pltpu.VMEM((IDXB, BLK), jnp.float32),
            pltpu.VMEM((ROWB, BLK, HIDDEN), jnp.float32),
        ]
        + [pltpu.SemaphoreType.DMA] * (IDXB + 2 * ROWB),
    )
    def seg(table, gidx, w, sidx, zrows, *rest):
        if teleport:
            h2a, out, acc, stab, gidx_r, sidx_r, w_r, rows_r, *sems = rest
        else:
            h2a = None
            out, acc, stab, gidx_r, sidx_r, w_r, rows_r, *sems = rest
        sem_i = sems[:IDXB]
        sem_g = sems[IDXB:IDXB + ROWB]
        sem_s = sems[IDXB + ROWB:]
        cid = lax.axis_index("c")
        sid = lax.axis_index("s")
        wid = cid * NS + sid
        rslice = pl.ds(sid * ROWS_PER_TILE, ROWS_PER_TILE)

        def fire_idx(b, j):
            pltpu.async_copy(gidx.at[wid, b], gidx_r.at[j], sem_i[j])
            pltpu.async_copy(sidx.at[wid, b], sidx_r.at[j], sem_i[j])
            pltpu.async_copy(w.at[wid, b], w_r.at[j], sem_i[j])

        def wait_idx(j):
            pltpu.make_async_copy(gidx.at[0, 0], gidx_r.at[j], sem_i[j]).wait()
            pltpu.make_async_copy(sidx.at[0, 0], sidx_r.at[j], sem_i[j]).wait()
            pltpu.make_async_copy(w.at[0, 0], w_r.at[j], sem_i[j]).wait()

        def fire_gather(ji, jr):
            pltpu.async_copy(stab.at[gidx_r.at[ji]], rows_r.at[jr], sem_g[jr])

        def wait_gather(jr):
            pltpu.make_async_copy(
                stab.at[gidx_r.at[0]], rows_r.at[jr], sem_g[jr]).wait()

        def fire_scatter(ji, jr):
            pltpu.async_copy(
                rows_r.at[jr], acc.at[sidx_r.at[ji]], sem_s[jr], add=True)

        def wait_scatter(jr):
            pltpu.make_async_copy(
                rows_r.at[jr], acc.at[sidx_r.at[0]], sem_s[jr]).wait()

        for b in range(LOOK_I):
            fire_idx(b, b)

        if teleport:
            @pl.when(cid == 0)
            def _():
                pltpu.async_copy(h2a.at[rslice], acc.at[rslice], sem_s[0])

            @pl.when(cid != 0)
            def _():
                pltpu.async_copy(zrows, acc.at[rslice], sem_s[0])
        else:
            pltpu.async_copy(zrows, acc.at[rslice], sem_s[0])

        if dual:
            nchunk = ROWS_PER_TILE // BLK

            def fire_stage(c):
                s0 = (2 * c) % ROWB
                base = sid * ROWS_PER_TILE + c * BLK
                pltpu.async_copy(table.at[0, pl.ds(base, BLK)],
                                 rows_r.at[s0], sem_g[s0])
                pltpu.async_copy(table.at[1, pl.ds(base, BLK)],
                                 rows_r.at[s0 + 1], sem_g[s0 + 1])

            fire_stage(0)
            for c in range(nchunk):
                if c + 1 < nchunk:
                    fire_stage(c + 1)
                s0 = (2 * c) % ROWB
                base = sid * ROWS_PER_TILE + c * BLK
                pltpu.make_async_copy(table.at[0, pl.ds(0, BLK)],
                                      rows_r.at[s0], sem_g[s0]).wait()
                pltpu.make_async_copy(table.at[0, pl.ds(0, BLK)],
                                      rows_r.at[s0 + 1], sem_g[s0 + 1]).wait()

                @plsc.parallel_loop(0, BLK, unroll=4)
                def _(e):
                    for g in range(CGRP):
                        sl = pl.ds(g * LANES, LANES)
                        rows_r[s0, e, sl] = (rows_r[s0, e, sl]
                                             + rows_r[s0 + 1, e, sl])

                pltpu.sync_copy(rows_r.at[s0], stab.at[pl.ds(base, BLK)])
        else:
            pltpu.sync_copy(
                table.at[pl.ds(sid * (trows // NS), trows // NS)],
                stab.at[pl.ds(sid * (trows // NS), trows // NS)])
        pltpu.make_async_copy(zrows, acc.at[rslice], sem_s[0]).wait()
        plsc.subcore_barrier()

        for b in range(LOOK_G):
            wait_idx(b)
            fire_gather(b, b)

        def super_body(s8, carry):
            b0 = s8 * IDXB
            for u in range(IDXB):
                b = b0 + u
                ji = u % IDXB
                jr = u % ROWB
                wait_gather(jr)

                @plsc.parallel_loop(0, BLK, unroll=4)
                def _(e):
                    wv = plsc.load_gather(
                        w_r,
                        [jnp.full((LANES,), ji, jnp.int32),
                         jnp.broadcast_to(e, (LANES,)).astype(jnp.int32)])
                    for g in range(CGRP):
                        sl = pl.ds(g * LANES, LANES)
                        rows_r[jr, e, sl] = rows_r[jr, e, sl] * wv

                fire_scatter(ji, jr)

                bi = b + LOOK_I
                jii = (u + LOOK_I) % IDXB

                @pl.when(bi < nblk)
                def _():
                    fire_idx(bi, jii)

                bg = b + LOOK_G
                jgi = (u + LOOK_G) % IDXB
                jgr = (u + LOOK_G) % ROWB

                @pl.when(bg < nblk)
                def _():
                    @pl.when(b >= ROWB - LOOK_G)
                    def _():
                        wait_scatter(jgr)
                    wait_idx(jgi)
                    fire_gather(jgi, jgr)

            return carry

        lax.fori_loop(0, nblk // IDXB, super_body, 0)
        for jr in range(ROWB):
            wait_scatter(jr)
        plsc.subcore_barrier()
        pltpu.sync_copy(acc.at[rslice], out.at[cid, rslice])

    return seg


_GRAN = NW * BLK * IDXB
_E1_PAD = ((500000 + _GRAN - 1) // _GRAN) * _GRAN
_E2_PAD = ((320000 + _GRAN - 1) // _GRAN) * _GRAN
_SEG1 = _make_seg_kernel(_E1_PAD, N_NODES, dual=False, teleport=False)
_SEG2F = _make_seg_kernel(_E2_PAD, NPAD, dual=False, teleport=True)
_SEG2 = _make_seg_kernel(_E2_PAD, NPAD, dual=True, teleport=True)


def _pad_lists(e_pad, gidx, w, sidx):
    pad = e_pad - gidx.shape[0]
    nblk = e_pad // NW // BLK
    return (
        jnp.pad(gidx, (0, pad)).reshape(NW, nblk, BLK),
        jnp.pad(w, (0, pad)).reshape(NW, nblk, BLK),
        jnp.pad(sidx, (0, pad)).reshape(NW, nblk, BLK),
    )


_R = 2048


def _mlp_body(p_ref, b1_ref, w2_ref, b2_ref, h2_ref, h2a_ref):
    h = jnp.maximum(p_ref[0] + p_ref[1] + b1_ref[...], 0.0)
    y = (jnp.dot(h, w2_ref[...], preferred_element_type=jnp.float32)
         + b2_ref[...])
    h2_ref[...] = y
    h2a_ref[...] = ALPHA * y


def _mlp(p, b1, W2, b2):
    return pl.pallas_call(
        _mlp_body,
        grid=(NPAD // _R,),
        in_specs=[
            pl.BlockSpec((NC, _R, HIDDEN), lambda i: (0, i, 0)),
            pl.BlockSpec((1, HIDDEN), lambda i: (0, 0)),
            pl.BlockSpec((HIDDEN, N_LABELS), lambda i: (0, 0)),
            pl.BlockSpec((1, N_LABELS), lambda i: (0, 0)),
        ],
        out_specs=[
            pl.BlockSpec((_R, N_LABELS), lambda i: (i, 0)),
            pl.BlockSpec((_R, N_LABELS), lambda i: (i, 0)),
        ],
        out_shape=[
            jax.ShapeDtypeStruct((NPAD, N_LABELS), jnp.float32),
            jax.ShapeDtypeStruct((NPAD, N_LABELS), jnp.float32),
        ],
    )(p, b1.reshape(1, HIDDEN), W2, b2.reshape(1, N_LABELS))


_RL = 2000


def _ls_body(q_ref, o_ref):
    t = q_ref[0] + q_ref[1]
    m = jnp.max(t, axis=1, keepdims=True)
    e = jnp.exp(t - m)
    o_ref[...] = t - m - jnp.log(jnp.sum(e, axis=1, keepdims=True))


def _ls(q):
    return pl.pallas_call(
        _ls_body,
        grid=(N_NODES // _RL,),
        in_specs=[pl.BlockSpec((NC, _RL, N_LABELS), lambda i: (0, i, 0))],
        out_specs=pl.BlockSpec((_RL, N_LABELS), lambda i: (i, 0)),
        out_shape=jax.ShapeDtypeStruct((N_NODES, N_LABELS), jnp.float32),
    )(q)


def kernel(feature_indices, feature_values, edge_indices, edge_weights,
           W1, b1, W2, b2):
    zrows = jnp.zeros((ROWS_PER_TILE, HIDDEN), jnp.float32)
    fg, fw, fs = _pad_lists(_E1_PAD, feature_indices[1], feature_values,
                            feature_indices[0])
    p = _SEG1(W1, fg, fw, fs, zrows)
    h2, h2a = _mlp(p, b1, W2, b2)
    eg, ew9, es = _pad_lists(_E2_PAD, edge_indices[1],
                             edge_weights * (1.0 - ALPHA), edge_indices[0])
    prev = _SEG2F(h2, eg, ew9, es, zrows, h2a)
    for _ in range(ITERATIONS_ - 1):
        prev = _SEG2(prev, eg, ew9, es, zrows, h2a)
    return _ls(prev)


# speedup vs baseline: 1.2010x; 1.2010x over previous
import functools

import jax
import jax.numpy as jnp
from jax import lax
from jax.experimental import pallas as pl
from jax.experimental.pallas import tpu as pltpu
from jax.experimental.pallas import tpu_sc as plsc

N_NODES = 10000
HIDDEN = 64
N_LABELS = 64
ALPHA = 0.1
ITERATIONS_ = 10

NC = 2
NS = 16
NW = NC * NS
BLK = 128
NPAD = 10240
ROWS_PER_TILE = NPAD // NS
LANES = 16
CGRP = HIDDEN // LANES


ROWB = 4
IDXB = 8
LOOK_G = 2
LOOK_I = 4


def _make_seg_kernel(e_pad, trows, dual, teleport):
    ew = e_pad // NW
    nblk = ew // BLK
    assert nblk % IDXB == 0
    mesh = plsc.VectorSubcoreMesh(core_axis_name="c", subcore_axis_name="s")

    @functools.partial(
        pl.kernel,
        mesh=mesh,
        compiler_params=pltpu.CompilerParams(
            needs_layout_passes=False, use_tc_tiling_on_sc=False),
        out_type=jax.ShapeDtypeStruct((NC, NPAD, HIDDEN), jnp.float32),
        scratch_types=[
            pltpu.VMEM_SHARED((NPAD, HIDDEN), jnp.float32),
            pltpu.VMEM_SHARED((trows, HIDDEN), jnp.float32),
            pltpu.VMEM((IDXB, BLK), jnp.int32),
            pltpu.VMEM((IDXB, BLK), jnp.int32),
            pltpu.VMEM((IDXB, BLK), jnp.float32),
            pltpu.VMEM((ROWB, BLK, HIDDEN), jnp.float32),
        ]
        + [pltpu.SemaphoreType.DMA] * (IDXB + 2 * ROWB),
    )
    def seg(table, gidx, w, sidx, zrows, *rest):
        if teleport:
            h2a, out, acc, stab, gidx_r, sidx_r, w_r, rows_r, *sems = rest
        else:
            h2a = None
            out, acc, stab, gidx_r, sidx_r, w_r, rows_r, *sems = rest
        sem_i = sems[:IDXB]
        sem_g = sems[IDXB:IDXB + ROWB]
        sem_s = sems[IDXB + ROWB:]
        cid = lax.axis_index("c")
        sid = lax.axis_index("s")
        wid = cid * NS + sid
        rslice = pl.ds(sid * ROWS_PER_TILE, ROWS_PER_TILE)

        def fire_idx(b, j):
            pltpu.async_copy(gidx.at[wid, b], gidx_r.at[j], sem_i[j])
            pltpu.async_copy(sidx.at[wid, b], sidx_r.at[j], sem_i[j])
            pltpu.async_copy(w.at[wid, b], w_r.at[j], sem_i[j])

        def wait_idx(j):
            pltpu.make_async_copy(gidx.at[0, 0], gidx_r.at[j], sem_i[j]).wait()
            pltpu.make_async_copy(sidx.at[0, 0], sidx_r.at[j], sem_i[j]).wait()
            pltpu.make_async_copy(w.at[0, 0], w_r.at[j], sem_i[j]).wait()

        def fire_gather(ji, jr):
            pltpu.async_copy(stab.at[gidx_r.at[ji]], rows_r.at[jr], sem_g[jr])

        def wait_gather(jr):
            pltpu.make_async_copy(
                stab.at[gidx_r.at[0]], rows_r.at[jr], sem_g[jr]).wait()

        def fire_scatter(ji, jr):
            pltpu.async_copy(
                rows_r.at[jr], acc.at[sidx_r.at[ji]], sem_s[jr], add=True)

        def wait_scatter(jr):
            pltpu.make_async_copy(
                rows_r.at[jr], acc.at[sidx_r.at[0]], sem_s[jr]).wait()

        for b in range(LOOK_I):
            fire_idx(b, b)

        if teleport:
            @pl.when(cid == 0)
            def _():
                pltpu.async_copy(h2a.at[rslice], acc.at[rslice], sem_s[0])

            @pl.when(cid != 0)
            def _():
                pltpu.async_copy(zrows, acc.at[rslice], sem_s[0])
        else:
            pltpu.async_copy(zrows, acc.at[rslice], sem_s[0])

        if dual:
            nchunk = ROWS_PER_TILE // BLK

            def fire_stage(c):
                s0 = (2 * c) % ROWB
                base = sid * ROWS_PER_TILE + c * BLK
                pltpu.async_copy(table.at[0, pl.ds(base, BLK)],
                                 rows_r.at[s0], sem_g[s0])
                pltpu.async_copy(table.at[1, pl.ds(base, BLK)],
                                 rows_r.at[s0 + 1], sem_g[s0 + 1])

            fire_stage(0)
            for c in range(nchunk):
                if c + 1 < nchunk:
                    fire_stage(c + 1)
                s0 = (2 * c) % ROWB
                base = sid * ROWS_PER_TILE + c * BLK
                pltpu.make_async_copy(table.at[0, pl.ds(0, BLK)],
                                      rows_r.at[s0], sem_g[s0]).wait()
                pltpu.make_async_copy(table.at[0, pl.ds(0, BLK)],
                                      rows_r.at[s0 + 1], sem_g[s0 + 1]).wait()

                @plsc.parallel_loop(0, BLK, unroll=4)
                def _(e):
                    for g in range(CGRP):
                        sl = pl.ds(g * LANES, LANES)
                        rows_r[s0, e, sl] = (rows_r[s0, e, sl]
                                             + rows_r[s0 + 1, e, sl])

                pltpu.sync_copy(rows_r.at[s0], stab.at[pl.ds(base, BLK)])
        else:
            pltpu.sync_copy(
                table.at[pl.ds(sid * (trows // NS), trows // NS)],
                stab.at[pl.ds(sid * (trows // NS), trows // NS)])
        pltpu.make_async_copy(zrows, acc.at[rslice], sem_s[0]).wait()
        plsc.subcore_barrier()

        for b in range(LOOK_G):
            wait_idx(b)
            fire_gather(b, b)

        def super_body(s8, carry):
            b0 = s8 * IDXB
            for u in range(IDXB):
                b = b0 + u
                ji = u % IDXB
                jr = u % ROWB
                wait_gather(jr)

                @plsc.parallel_loop(0, BLK, unroll=4)
                def _(e):
                    wv = plsc.load_gather(
                        w_r,
                        [jnp.full((LANES,), ji, jnp.int32),
                         jnp.broadcast_to(e, (LANES,)).astype(jnp.int32)])
                    for g in range(CGRP):
                        sl = pl.ds(g * LANES, LANES)
                        rows_r[jr, e, sl] = rows_r[jr, e, sl] * wv

                fire_scatter(ji, jr)

                bi = b + LOOK_I
                jii = (u + LOOK_I) % IDXB

                @pl.when(bi < nblk)
                def _():
                    fire_idx(bi, jii)

                bg = b + LOOK_G
                jgi = (u + LOOK_G) % IDXB
                jgr = (u + LOOK_G) % ROWB

                @pl.when(bg < nblk)
                def _():
                    @pl.when(b >= ROWB - LOOK_G)
                    def _():
                        wait_scatter(jgr)
                    wait_idx(jgi)
                    fire_gather(jgi, jgr)

            return carry

        lax.fori_loop(0, nblk // IDXB, super_body, 0)
        for jr in range(ROWB):
            wait_scatter(jr)
        plsc.subcore_barrier()
        pltpu.sync_copy(acc.at[rslice], out.at[cid, rslice])

    return seg


_GRAN = NW * BLK * IDXB
_E1_PAD = ((500000 + _GRAN - 1) // _GRAN) * _GRAN
_E2_PAD = ((320000 + _GRAN - 1) // _GRAN) * _GRAN
_SEG1 = _make_seg_kernel(_E1_PAD, N_NODES, dual=False, teleport=False)
_SEG2F = _make_seg_kernel(_E2_PAD, NPAD, dual=False, teleport=True)
_SEG2 = _make_seg_kernel(_E2_PAD, NPAD, dual=True, teleport=True)


def _pad_lists(e_pad, gidx, w, sidx):
    pad = e_pad - gidx.shape[0]
    nblk = e_pad // NW // BLK
    return (
        jnp.pad(gidx, (0, pad)).reshape(NW, nblk, BLK),
        jnp.pad(w, (0, pad)).reshape(NW, nblk, BLK),
        jnp.pad(sidx, (0, pad)).reshape(NW, nblk, BLK),
    )


_R = 2048


def _mlp_body(p_ref, b1_ref, w2_ref, b2_ref, h2_ref, h2a_ref):
    h = jnp.maximum(p_ref[0] + p_ref[1] + b1_ref[...], 0.0)
    y = (jnp.dot(h, w2_ref[...], preferred_element_type=jnp.float32)
         + b2_ref[...])
    h2_ref[...] = y
    h2a_ref[...] = ALPHA * y


def _mlp(p, b1, W2, b2):
    return pl.pallas_call(
        _mlp_body,
        grid=(NPAD // _R,),
        in_specs=[
            pl.BlockSpec((NC, _R, HIDDEN), lambda i: (0, i, 0)),
            pl.BlockSpec((1, HIDDEN), lambda i: (0, 0)),
            pl.BlockSpec((HIDDEN, N_LABELS), lambda i: (0, 0)),
            pl.BlockSpec((1, N_LABELS), lambda i: (0, 0)),
        ],
        out_specs=[
            pl.BlockSpec((_R, N_LABELS), lambda i: (i, 0)),
            pl.BlockSpec((_R, N_LABELS), lambda i: (i, 0)),
        ],
        out_shape=[
            jax.ShapeDtypeStruct((NPAD, N_LABELS), jnp.float32),
            jax.ShapeDtypeStruct((NPAD, N_LABELS), jnp.float32),
        ],
    )(p, b1.reshape(1, HIDDEN), W2, b2.reshape(1, N_LABELS))


_RL = 2000


def _ls_body(q_ref, o_ref):
    t = q_ref[0] + q_ref[1]
    m = jnp.max(t, axis=1, keepdims=True)
    e = jnp.exp(t - m)
    o_ref[...] = t - m - jnp.log(jnp.sum(e, axis=1, keepdims=True))


def _ls(q):
    return pl.pallas_call(
        _ls_body,
        grid=(N_NODES // _RL,),
        in_specs=[pl.BlockSpec((NC, _RL, N_LABELS), lambda i: (0, i, 0))],
        out_specs=pl.BlockSpec((_RL, N_LABELS), lambda i: (i, 0)),
        out_shape=jax.ShapeDtypeStruct((N_NODES, N_LABELS), jnp.float32),
    )(q)


def kernel(feature_indices, feature_values, edge_indices, edge_weights,
           W1, b1, W2, b2):
    zrows = jnp.zeros((ROWS_PER_TILE, HIDDEN), jnp.float32)
    fg, fw, fs = _pad_lists(_E1_PAD, feature_indices[1], feature_values,
                            feature_indices[0])
    p = _SEG1(W1, fg, fw, fs, zrows)
    h2, h2a = _mlp(p, b1, W2, b2)
    eg, ew9, es = _pad_lists(_E2_PAD, edge_indices[1],
                             edge_weights * (1.0 - ALPHA), edge_indices[0])
    prev = _SEG2F(h2, eg, ew9, es, zrows, h2a)
    for _ in range(ITERATIONS_ - 1):
        prev = _SEG2(prev, eg, ew9, es, zrows, h2a)
    return _ls(prev)
